# baseline (speedup 1.0000x reference)
import functools

import jax
import jax.numpy as jnp
from jax import lax
from jax.experimental import pallas as pl
from jax.experimental.pallas import tpu as pltpu
from jax.experimental.pallas import tpu_sc as plsc


def kernel(x, emb_weight):
    seq_len = x.shape[1]
    dim = emb_weight.shape[1]
    info = plsc.get_sparse_core_info()
    nw = info.num_cores * info.num_subcores
    rows_per_w = seq_len // nw
    mesh = plsc.VectorSubcoreMesh(core_axis_name="c", subcore_axis_name="s")

    @functools.partial(
        pl.kernel,
        mesh=mesh,
        out_type=jax.ShapeDtypeStruct((seq_len, dim), emb_weight.dtype),
    )
    def copy_k(table_hbm, out_hbm):
        wid = lax.axis_index("s") * info.num_cores + lax.axis_index("c")
        base = wid * rows_per_w
        pltpu.sync_copy(
            table_hbm.at[pl.ds(base, rows_per_w)],
            out_hbm.at[pl.ds(base, rows_per_w)],
        )

    return copy_k(emb_weight)[None, :, :]


# speedup vs baseline: 16.7385x; 16.7385x over previous
import functools

import jax
import jax.numpy as jnp
from jax import lax
from jax.experimental import pallas as pl
from jax.experimental.pallas import tpu as pltpu
from jax.experimental.pallas import tpu_sc as plsc

_CHUNK_ROWS = 32


def kernel(x, emb_weight):
    seq_len = x.shape[1]
    dim = emb_weight.shape[1]
    info = plsc.get_sparse_core_info()
    nw = info.num_cores * info.num_subcores
    rows_per_w = seq_len // nw
    nchunks = rows_per_w // _CHUNK_ROWS
    mesh = plsc.VectorSubcoreMesh(core_axis_name="c", subcore_axis_name="s")

    @functools.partial(
        pl.kernel,
        mesh=mesh,
        out_type=jax.ShapeDtypeStruct((seq_len, dim), emb_weight.dtype),
        scratch_types=[
            pltpu.VMEM((2, _CHUNK_ROWS, dim), emb_weight.dtype),
            pltpu.SemaphoreType.DMA,
            pltpu.SemaphoreType.DMA,
            pltpu.SemaphoreType.DMA,
            pltpu.SemaphoreType.DMA,
        ],
    )
    def copy_k(table_hbm, out_hbm, buf, in0, in1, out0, out1):
        wid = lax.axis_index("s") * info.num_cores + lax.axis_index("c")
        base = wid * rows_per_w
        in_sems = (in0, in1)
        out_sems = (out0, out1)

        def gather(i):
            return pltpu.async_copy(
                table_hbm.at[pl.ds(base + i * _CHUNK_ROWS, _CHUNK_ROWS)],
                buf.at[i % 2],
                in_sems[i % 2],
            )

        def scatter(i):
            return pltpu.async_copy(
                buf.at[i % 2],
                out_hbm.at[pl.ds(base + i * _CHUNK_ROWS, _CHUNK_ROWS)],
                out_sems[i % 2],
            )

        gathers = [None] * nchunks
        scatters = [None] * nchunks
        gathers[0] = gather(0)
        for i in range(nchunks):
            if i + 1 < nchunks:
                if i >= 1:
                    scatters[i - 1].wait()
                gathers[i + 1] = gather(i + 1)
            gathers[i].wait()
            scatters[i] = scatter(i)
        scatters[nchunks - 2].wait()
        scatters[nchunks - 1].wait()

    return copy_k(emb_weight)[None, :, :]


# speedup vs baseline: 17.0987x; 1.0215x over previous
import functools

import jax
import jax.numpy as jnp
from jax import lax
from jax.experimental import pallas as pl
from jax.experimental.pallas import tpu as pltpu
from jax.experimental.pallas import tpu_sc as plsc

_CHUNK_ROWS = 16
_NBUF = 4


def kernel(x, emb_weight):
    seq_len = x.shape[1]
    dim = emb_weight.shape[1]
    info = plsc.get_sparse_core_info()
    nw = info.num_cores * info.num_subcores
    rows_per_w = seq_len // nw
    nchunks = rows_per_w // _CHUNK_ROWS
    mesh = plsc.VectorSubcoreMesh(core_axis_name="c", subcore_axis_name="s")

    @functools.partial(
        pl.kernel,
        mesh=mesh,
        out_type=jax.ShapeDtypeStruct((seq_len, dim), emb_weight.dtype),
        scratch_types=[
            pltpu.VMEM((_NBUF, _CHUNK_ROWS, dim), emb_weight.dtype),
        ]
        + [pltpu.SemaphoreType.DMA] * (2 * _NBUF),
    )
    def copy_k(table_hbm, out_hbm, buf, *sems):
        wid = lax.axis_index("s") * info.num_cores + lax.axis_index("c")
        base = wid * rows_per_w
        in_sems = sems[:_NBUF]
        out_sems = sems[_NBUF:]

        def gather(i):
            return pltpu.async_copy(
                table_hbm.at[pl.ds(base + i * _CHUNK_ROWS, _CHUNK_ROWS)],
                buf.at[i % _NBUF],
                in_sems[i % _NBUF],
            )

        def scatter(i):
            return pltpu.async_copy(
                buf.at[i % _NBUF],
                out_hbm.at[pl.ds(base + i * _CHUNK_ROWS, _CHUNK_ROWS)],
                out_sems[i % _NBUF],
            )

        gathers = [None] * nchunks
        scatters = [None] * nchunks
        for i in range(min(_NBUF - 1, nchunks)):
            gathers[i] = gather(i)
        for i in range(nchunks):
            j = i + _NBUF - 1
            if j < nchunks:
                if j >= _NBUF:
                    scatters[j - _NBUF].wait()
                gathers[j] = gather(j)
            gathers[i].wait()
            scatters[i] = scatter(i)
        for i in range(max(0, nchunks - _NBUF), nchunks):
            scatters[i].wait()

    return copy_k(emb_weight)[None, :, :]
